# Initial kernel scaffold; baseline (speedup 1.0000x reference)
import jax
import jax.numpy as jnp
from jax.experimental import pallas as pl


def kernel(g, h, W, b):
    raise NotImplementedError("write your pallas kernel here")


# speedup vs baseline: 1.6826x; 1.6826x over previous
import functools

import jax
import jax.numpy as jnp
from jax import lax
from jax.experimental import pallas as pl
from jax.experimental.pallas import tpu as pltpu
from jax.experimental.pallas import tpu_sc as plsc

N = 4096
D = 256
K = 2048

_RANK_BLK = 512
_MM_KBLK = 512
_NORM_BLK = 512


def _rank_body(srow_ref, scol_ref, h_ref, idx_ref, hs_ref):
    i = pl.program_id(0)

    @pl.when(i == 0)
    def _():
        idx_ref[...] = jnp.zeros_like(idx_ref)

    sc = scol_ref[...]
    sr = srow_ref[...]
    jj = lax.broadcasted_iota(jnp.int32, (_RANK_BLK, N), 1)
    ii = lax.broadcasted_iota(jnp.int32, (_RANK_BLK, 1), 0) + i * _RANK_BLK
    beats = (sr > sc) | ((sr == sc) & (jj < ii))
    rank = jnp.sum(beats.astype(jnp.int32), axis=1, keepdims=True)
    pp = lax.broadcasted_iota(jnp.int32, (_RANK_BLK, K), 1)
    hit = rank == pp
    idx_ref[...] += jnp.sum(jnp.where(hit, ii, 0), axis=0, keepdims=True)
    hs_ref[...] = h_ref[...] * sc


def _rank_call(s_row, s_col, h):
    nblk = N // _RANK_BLK
    return pl.pallas_call(
        _rank_body,
        grid=(nblk,),
        in_specs=[
            pl.BlockSpec((1, N), lambda i: (0, 0)),
            pl.BlockSpec((_RANK_BLK, 1), lambda i: (i, 0)),
            pl.BlockSpec((_RANK_BLK, D), lambda i: (i, 0)),
        ],
        out_specs=(
            pl.BlockSpec((1, K), lambda i: (0, 0)),
            pl.BlockSpec((_RANK_BLK, D), lambda i: (i, 0)),
        ),
        out_shape=(
            jax.ShapeDtypeStruct((1, K), jnp.int32),
            jax.ShapeDtypeStruct((N, D), jnp.float32),
        ),
    )(s_row, s_col, h)


def _tr_body(g_ref, at_ref):
    at_ref[...] = (jnp.transpose(g_ref[...]) != 0).astype(jnp.float32)


def _tr_call(g):
    blk = 512
    nblk = N // blk
    return pl.pallas_call(
        _tr_body,
        grid=(nblk, nblk),
        in_specs=[pl.BlockSpec((blk, blk), lambda i, j: (i, j))],
        out_specs=pl.BlockSpec((blk, blk), lambda i, j: (j, i)),
        out_shape=jax.ShapeDtypeStruct((N, N), jnp.float32),
    )(g)


@functools.lru_cache(maxsize=None)
def _make_sc_gather(width, batch, rows_per_dma):
    info = plsc.get_sparse_core_info()
    nc, ns = info.num_cores, info.num_subcores
    nw = nc * ns
    b_per_w = batch // nw
    n_dma = b_per_w // rows_per_dma
    mesh = plsc.VectorSubcoreMesh(core_axis_name="c", subcore_axis_name="s")

    @functools.partial(
        pl.kernel,
        mesh=mesh,
        out_type=jax.ShapeDtypeStruct((batch, width), jnp.float32),
        scratch_types=[
            pltpu.VMEM((b_per_w,), jnp.int32),
            pltpu.VMEM((rows_per_dma, width), jnp.float32),
            pltpu.SemaphoreType.DMA,
        ],
    )
    def k(table_hbm, idx_hbm, out_hbm, idx_v, rows_v, sem):
        wid = lax.axis_index("s") * nc + lax.axis_index("c")
        base = wid * b_per_w
        pltpu.sync_copy(idx_hbm.at[pl.ds(base, b_per_w)], idx_v)
        for j in range(n_dma):
            sl = idx_v.at[pl.ds(j * rows_per_dma, rows_per_dma)]
            pltpu.async_copy(table_hbm.at[sl], rows_v, sem).wait()
            pltpu.sync_copy(
                rows_v, out_hbm.at[pl.ds(base + j * rows_per_dma, rows_per_dma)]
            )

    return k


def _mm_body(ar_ref, act_ref, u_ref, deg_ref):
    kk = pl.program_id(0)

    @pl.when(kk == 0)
    def _():
        u_ref[...] = jnp.zeros_like(u_ref)

    a = (ar_ref[...] != 0).astype(jnp.bfloat16)
    c = act_ref[...].astype(jnp.bfloat16)
    u_ref[...] += lax.dot_general(
        a, c, (((1,), (1,)), ((), ())), preferred_element_type=jnp.float32
    )

    @pl.when(kk == pl.num_programs(0) - 1)
    def _():
        u = (u_ref[...] != 0).astype(jnp.float32)
        u_ref[...] = u
        deg_ref[...] = jnp.sum(u, axis=1, keepdims=True)


def _mm_call(ar, act):
    nblk = N // _MM_KBLK
    return pl.pallas_call(
        _mm_body,
        grid=(nblk,),
        in_specs=[
            pl.BlockSpec((K, _MM_KBLK), lambda k: (0, k)),
            pl.BlockSpec((K, _MM_KBLK), lambda k: (0, k)),
        ],
        out_specs=(
            pl.BlockSpec((K, K), lambda k: (0, 0)),
            pl.BlockSpec((K, 1), lambda k: (0, 0)),
        ),
        out_shape=(
            jax.ShapeDtypeStruct((K, K), jnp.float32),
            jax.ShapeDtypeStruct((K, 1), jnp.float32),
        ),
    )(ar, act)


def _norm_body(u_ref, dcol_ref, drow_ref, out_ref):
    i = pl.program_id(0)
    dcol = lax.rsqrt(dcol_ref[...])
    drow = lax.rsqrt(drow_ref[...])
    ii = lax.broadcasted_iota(jnp.int32, (_NORM_BLK, K), 0) + i * _NORM_BLK
    jj = lax.broadcasted_iota(jnp.int32, (_NORM_BLK, K), 1)
    eye = (ii == jj).astype(jnp.float32)
    out_ref[...] = u_ref[...] * dcol * drow + eye


def _norm_call(u, deg, deg_row):
    nblk = K // _NORM_BLK
    return pl.pallas_call(
        _norm_body,
        grid=(nblk,),
        in_specs=[
            pl.BlockSpec((_NORM_BLK, K), lambda i: (i, 0)),
            pl.BlockSpec((_NORM_BLK, 1), lambda i: (i, 0)),
            pl.BlockSpec((1, K), lambda i: (0, 0)),
        ],
        out_specs=pl.BlockSpec((_NORM_BLK, K), lambda i: (i, 0)),
        out_shape=jax.ShapeDtypeStruct((K, K), jnp.float32),
    )(u, deg, deg_row)


def kernel(g, h, W, b):
    _gather_h = _make_sc_gather(D, K, 64)
    _gather_wide = _make_sc_gather(N, K, 8)
    scores = jax.nn.sigmoid(jnp.squeeze(h @ W + b, -1))
    idx2d, hs = _rank_call(scores.reshape(1, N), scores.reshape(N, 1), h)
    idx = idx2d.reshape(K)
    at = _tr_call(g)
    new_h = _gather_h(hs, idx)
    ar = _gather_wide(g, idx)
    act = _gather_wide(at, idx)
    u, deg = _mm_call(ar, act)
    g_new = _norm_call(u, deg, deg.reshape(1, K))
    return (g_new, new_h, idx)


# speedup vs baseline: 1.8313x; 1.0884x over previous
import functools

import jax
import jax.numpy as jnp
from jax import lax
from jax.experimental import pallas as pl
from jax.experimental.pallas import tpu as pltpu
from jax.experimental.pallas import tpu_sc as plsc

N = 4096
D = 256
K = 2048

_RANK_BLK = 512
_MM_KBLK = 512
_NORM_BLK = 512


def _rank_body(srow_ref, scol_ref, h_ref, idx_ref, hs_ref):
    i = pl.program_id(0)

    @pl.when(i == 0)
    def _():
        idx_ref[...] = jnp.zeros_like(idx_ref)

    sc = scol_ref[...]
    sr = srow_ref[...]
    jj = lax.broadcasted_iota(jnp.int32, (_RANK_BLK, N), 1)
    ii = lax.broadcasted_iota(jnp.int32, (_RANK_BLK, 1), 0) + i * _RANK_BLK
    beats = (sr > sc) | ((sr == sc) & (jj < ii))
    rank = jnp.sum(beats.astype(jnp.int32), axis=1, keepdims=True)
    pp = lax.broadcasted_iota(jnp.int32, (_RANK_BLK, K), 1)
    hit = rank == pp
    idx_ref[...] += jnp.sum(jnp.where(hit, ii, 0), axis=0, keepdims=True)
    hs_ref[...] = h_ref[...] * sc


def _rank_call(s_row, s_col, h):
    nblk = N // _RANK_BLK
    return pl.pallas_call(
        _rank_body,
        grid=(nblk,),
        in_specs=[
            pl.BlockSpec((1, N), lambda i: (0, 0)),
            pl.BlockSpec((_RANK_BLK, 1), lambda i: (i, 0)),
            pl.BlockSpec((_RANK_BLK, D), lambda i: (i, 0)),
        ],
        out_specs=(
            pl.BlockSpec((1, K), lambda i: (0, 0)),
            pl.BlockSpec((_RANK_BLK, D), lambda i: (i, 0)),
        ),
        out_shape=(
            jax.ShapeDtypeStruct((1, K), jnp.int32),
            jax.ShapeDtypeStruct((N, D), jnp.float32),
        ),
    )(s_row, s_col, h)


def _tr_body(g_ref, at_ref):
    at_ref[...] = (jnp.transpose(g_ref[...]) != 0).astype(jnp.float32)


def _tr_call(g):
    blk = 512
    nblk = N // blk
    return pl.pallas_call(
        _tr_body,
        grid=(nblk, nblk),
        in_specs=[pl.BlockSpec((blk, blk), lambda i, j: (i, j))],
        out_specs=pl.BlockSpec((blk, blk), lambda i, j: (j, i)),
        out_shape=jax.ShapeDtypeStruct((N, N), jnp.float32),
    )(g)


@functools.lru_cache(maxsize=None)
def _make_sc_gather(width, batch, rows_per_dma, dtype):
    info = plsc.get_sparse_core_info()
    nc, ns = info.num_cores, info.num_subcores
    nw = nc * ns
    b_per_w = batch // nw
    n_dma = b_per_w // rows_per_dma
    mesh = plsc.VectorSubcoreMesh(core_axis_name="c", subcore_axis_name="s")

    @functools.partial(
        pl.kernel,
        mesh=mesh,
        out_type=jax.ShapeDtypeStruct((batch, width), dtype),
        scratch_types=[
            pltpu.VMEM((b_per_w,), jnp.int32),
            pltpu.VMEM((rows_per_dma, width), dtype),
            pltpu.VMEM((rows_per_dma, width), dtype),
            pltpu.SemaphoreType.DMA,
            pltpu.SemaphoreType.DMA,
        ],
    )
    def k(table_hbm, idx_hbm, out_hbm, idx_v, buf0, buf1, sem0, sem1):
        wid = lax.axis_index("s") * nc + lax.axis_index("c")
        base = wid * b_per_w
        pltpu.sync_copy(idx_hbm.at[pl.ds(base, b_per_w)], idx_v)
        bufs, sems, cps = (buf0, buf1), (sem0, sem1), [None, None]
        r = rows_per_dma
        cps[0] = pltpu.async_copy(table_hbm.at[idx_v.at[pl.ds(0, r)]], bufs[0], sems[0])
        for j in range(n_dma):
            cur, nxt = j % 2, (j + 1) % 2
            if j + 1 < n_dma:
                cps[nxt] = pltpu.async_copy(
                    table_hbm.at[idx_v.at[pl.ds((j + 1) * r, r)]], bufs[nxt], sems[nxt]
                )
            cps[cur].wait()
            pltpu.sync_copy(bufs[cur], out_hbm.at[pl.ds(base + j * r, r)])

    return k


def _mm_body(ar_ref, act_ref, out_ref):
    kk = pl.program_id(0)

    @pl.when(kk == 0)
    def _():
        out_ref[...] = jnp.zeros_like(out_ref)

    a = (ar_ref[...] != 0).astype(jnp.bfloat16)
    c = act_ref[...].astype(jnp.bfloat16)
    out_ref[...] += lax.dot_general(
        a, c, (((1,), (1,)), ((), ())), preferred_element_type=jnp.float32
    )

    @pl.when(kk == pl.num_programs(0) - 1)
    def _():
        u = (out_ref[...] != 0).astype(jnp.float32)
        deg = jnp.sum(u, axis=1, keepdims=True)
        dcol = lax.rsqrt(deg)
        drow = jnp.transpose(dcol)
        ii = lax.broadcasted_iota(jnp.int32, (K, K), 0)
        jj = lax.broadcasted_iota(jnp.int32, (K, K), 1)
        eye = (ii == jj).astype(jnp.float32)
        out_ref[...] = u * dcol * drow + eye


def _mm_call(ar, act):
    nblk = N // _MM_KBLK
    return pl.pallas_call(
        _mm_body,
        grid=(nblk,),
        in_specs=[
            pl.BlockSpec((K, _MM_KBLK), lambda k: (0, k)),
            pl.BlockSpec((K, _MM_KBLK), lambda k: (0, k)),
        ],
        out_specs=pl.BlockSpec((K, K), lambda k: (0, 0)),
        out_shape=jax.ShapeDtypeStruct((K, K), jnp.float32),
    )(ar, act)


def kernel(g, h, W, b):
    _gather_h = _make_sc_gather(D, K, 64, jnp.float32)
    _gather_g = _make_sc_gather(N, K, 8, jnp.float32)
    _gather_at = _make_sc_gather(N, K, 8, jnp.float32)
    scores = jax.nn.sigmoid(jnp.squeeze(h @ W + b, -1))
    idx2d, hs = _rank_call(scores.reshape(1, N), scores.reshape(N, 1), h)
    idx = idx2d.reshape(K)
    new_h = _gather_h(hs, idx)
    ar = _gather_g(g, idx)
    at = _tr_call(g)
    act = _gather_at(at, idx)
    g_new = _mm_call(ar, act)
    return (g_new, new_h, idx)


# speedup vs baseline: 2.3254x; 1.2698x over previous
import functools

import jax
import jax.numpy as jnp
from jax import lax
from jax.experimental import pallas as pl
from jax.experimental.pallas import tpu as pltpu
from jax.experimental.pallas import tpu_sc as plsc

N = 4096
D = 256
K = 2048

_RANK_BLK = 512
_MM_KBLK = 512
_NORM_BLK = 512


def _rank_body(srow_ref, scol_ref, h_ref, idx_ref, hs_ref):
    i = pl.program_id(0)

    @pl.when(i == 0)
    def _():
        idx_ref[...] = jnp.zeros_like(idx_ref)

    sc = scol_ref[...]
    sr = srow_ref[...]
    jj = lax.broadcasted_iota(jnp.int32, (_RANK_BLK, N), 1)
    ii = lax.broadcasted_iota(jnp.int32, (_RANK_BLK, 1), 0) + i * _RANK_BLK
    beats = (sr > sc) | ((sr == sc) & (jj < ii))
    rank = jnp.sum(beats.astype(jnp.int32), axis=1, keepdims=True)
    pp = lax.broadcasted_iota(jnp.int32, (_RANK_BLK, K), 1)
    hit = rank == pp
    idx_ref[...] += jnp.sum(jnp.where(hit, ii, 0), axis=0, keepdims=True)
    hs_ref[...] = h_ref[...] * sc


def _rank_call(s_row, s_col, h):
    nblk = N // _RANK_BLK
    return pl.pallas_call(
        _rank_body,
        grid=(nblk,),
        in_specs=[
            pl.BlockSpec((1, N), lambda i: (0, 0)),
            pl.BlockSpec((_RANK_BLK, 1), lambda i: (i, 0)),
            pl.BlockSpec((_RANK_BLK, D), lambda i: (i, 0)),
        ],
        out_specs=(
            pl.BlockSpec((1, K), lambda i: (0, 0)),
            pl.BlockSpec((_RANK_BLK, D), lambda i: (i, 0)),
        ),
        out_shape=(
            jax.ShapeDtypeStruct((1, K), jnp.int32),
            jax.ShapeDtypeStruct((N, D), jnp.float32),
        ),
    )(s_row, s_col, h)


def _colsel_body(idx_ref, g_ref, ac_ref, pt_ref):
    kk = pl.program_id(0)

    @pl.when(kk == 0)
    def _():
        mm = lax.broadcasted_iota(jnp.int32, (N, K), 0)
        pt_ref[...] = (idx_ref[...] == mm).astype(jnp.bfloat16)

    a = (g_ref[...] != 0).astype(jnp.bfloat16)
    ac_ref[...] = lax.dot_general(
        a, pt_ref[...], (((1,), (0,)), ((), ())),
        preferred_element_type=jnp.float32,
    ).astype(jnp.bfloat16)


def _colsel_call(idx2d, g):
    nblk = N // _MM_KBLK
    return pl.pallas_call(
        _colsel_body,
        grid=(nblk,),
        in_specs=[
            pl.BlockSpec((1, K), lambda k: (0, 0)),
            pl.BlockSpec((_MM_KBLK, N), lambda k: (k, 0)),
        ],
        out_specs=pl.BlockSpec((_MM_KBLK, K), lambda k: (k, 0)),
        out_shape=jax.ShapeDtypeStruct((N, K), jnp.bfloat16),
        scratch_shapes=[pltpu.VMEM((N, K), jnp.bfloat16)],
    )(idx2d, g)


@functools.lru_cache(maxsize=None)
def _make_sc_gather(width, batch, rows_per_dma, dtype):
    info = plsc.get_sparse_core_info()
    nc, ns = info.num_cores, info.num_subcores
    nw = nc * ns
    b_per_w = batch // nw
    n_dma = b_per_w // rows_per_dma
    mesh = plsc.VectorSubcoreMesh(core_axis_name="c", subcore_axis_name="s")

    @functools.partial(
        pl.kernel,
        mesh=mesh,
        out_type=jax.ShapeDtypeStruct((batch, width), dtype),
        scratch_types=[
            pltpu.VMEM((b_per_w,), jnp.int32),
            pltpu.VMEM((rows_per_dma, width), dtype),
            pltpu.VMEM((rows_per_dma, width), dtype),
            pltpu.SemaphoreType.DMA,
            pltpu.SemaphoreType.DMA,
        ],
    )
    def k(table_hbm, idx_hbm, out_hbm, idx_v, buf0, buf1, sem0, sem1):
        wid = lax.axis_index("s") * nc + lax.axis_index("c")
        base = wid * b_per_w
        pltpu.sync_copy(idx_hbm.at[pl.ds(base, b_per_w)], idx_v)
        bufs, sems, cps = (buf0, buf1), (sem0, sem1), [None, None]
        r = rows_per_dma
        cps[0] = pltpu.async_copy(table_hbm.at[idx_v.at[pl.ds(0, r)]], bufs[0], sems[0])
        for j in range(n_dma):
            cur, nxt = j % 2, (j + 1) % 2
            if j + 1 < n_dma:
                cps[nxt] = pltpu.async_copy(
                    table_hbm.at[idx_v.at[pl.ds((j + 1) * r, r)]], bufs[nxt], sems[nxt]
                )
            cps[cur].wait()
            pltpu.sync_copy(bufs[cur], out_hbm.at[pl.ds(base + j * r, r)])

    return k


def _mm_body(ar_ref, act_ref, out_ref):
    kk = pl.program_id(0)

    @pl.when(kk == 0)
    def _():
        out_ref[...] = jnp.zeros_like(out_ref)

    a = (ar_ref[...] != 0).astype(jnp.bfloat16)
    c = act_ref[...]
    out_ref[...] += lax.dot_general(
        a, c, (((1,), (0,)), ((), ())), preferred_element_type=jnp.float32
    )

    @pl.when(kk == pl.num_programs(0) - 1)
    def _():
        u = (out_ref[...] != 0).astype(jnp.float32)
        deg = jnp.sum(u, axis=1, keepdims=True)
        dcol = lax.rsqrt(deg)
        drow = jnp.transpose(dcol)
        ii = lax.broadcasted_iota(jnp.int32, (K, K), 0)
        jj = lax.broadcasted_iota(jnp.int32, (K, K), 1)
        eye = (ii == jj).astype(jnp.float32)
        out_ref[...] = u * dcol * drow + eye


def _mm_call(ar, act):
    nblk = N // _MM_KBLK
    return pl.pallas_call(
        _mm_body,
        grid=(nblk,),
        in_specs=[
            pl.BlockSpec((K, _MM_KBLK), lambda k: (0, k)),
            pl.BlockSpec((_MM_KBLK, K), lambda k: (k, 0)),
        ],
        out_specs=pl.BlockSpec((K, K), lambda k: (0, 0)),
        out_shape=jax.ShapeDtypeStruct((K, K), jnp.float32),
    )(ar, act)


def kernel(g, h, W, b):
    _gather_h = _make_sc_gather(D, K, 64, jnp.float32)
    _gather_g = _make_sc_gather(N, K, 8, jnp.float32)
    scores = jax.nn.sigmoid(jnp.squeeze(h @ W + b, -1))
    idx2d, hs = _rank_call(scores.reshape(1, N), scores.reshape(N, 1), h)
    idx = idx2d.reshape(K)
    new_h = _gather_h(hs, idx)
    ar = _gather_g(g, idx)
    ac = _colsel_call(idx2d, g)
    g_new = _mm_call(ar, ac)
    return (g_new, new_h, idx)


# speedup vs baseline: 2.9492x; 1.2683x over previous
import functools

import jax
import jax.numpy as jnp
from jax import lax
from jax.experimental import pallas as pl
from jax.experimental.pallas import tpu as pltpu
from jax.experimental.pallas import tpu_sc as plsc

N = 4096
D = 256
K = 2048

_RANK_BLK = 512
_MM_KBLK = 512
_NORM_BLK = 512


_PACK = 8
_BW = N // _PACK


def _rank_pack_body(srow_ref, scol_ref, h_ref, g_ref, idx_ref, hs_ref,
                    bits_ref, w8_ref):
    i = pl.program_id(0)

    @pl.when(i == 0)
    def _():
        idx_ref[...] = jnp.zeros_like(idx_ref)
        mi = lax.broadcasted_iota(jnp.int32, (N, _BW), 0)
        ci = lax.broadcasted_iota(jnp.int32, (N, _BW), 1)
        w8_ref[...] = jnp.where(
            (mi >> 3) == ci, 1 << (mi & 7), 0
        ).astype(jnp.bfloat16)

    sc = scol_ref[...]
    sr = srow_ref[...]
    jj = lax.broadcasted_iota(jnp.int32, (_RANK_BLK, N), 1)
    ii = lax.broadcasted_iota(jnp.int32, (_RANK_BLK, 1), 0) + i * _RANK_BLK
    beats = (sr > sc) | ((sr == sc) & (jj < ii))
    rank = jnp.sum(beats.astype(jnp.int32), axis=1, keepdims=True)
    pp = lax.broadcasted_iota(jnp.int32, (_RANK_BLK, K), 1)
    hit = rank == pp
    idx_ref[...] += jnp.sum(jnp.where(hit, ii, 0), axis=0, keepdims=True)
    hs_ref[...] = h_ref[...] * sc
    a = (g_ref[...] != 0).astype(jnp.bfloat16)
    bits_ref[...] = lax.dot_general(
        a, w8_ref[...], (((1,), (0,)), ((), ())),
        preferred_element_type=jnp.float32,
    )


def _rank_pack_call(s_row, s_col, h, g):
    nblk = N // _RANK_BLK
    return pl.pallas_call(
        _rank_pack_body,
        grid=(nblk,),
        in_specs=[
            pl.BlockSpec((1, N), lambda i: (0, 0)),
            pl.BlockSpec((_RANK_BLK, 1), lambda i: (i, 0)),
            pl.BlockSpec((_RANK_BLK, D), lambda i: (i, 0)),
            pl.BlockSpec((_RANK_BLK, N), lambda i: (i, 0)),
        ],
        out_specs=(
            pl.BlockSpec((1, K), lambda i: (0, 0)),
            pl.BlockSpec((_RANK_BLK, D), lambda i: (i, 0)),
            pl.BlockSpec((_RANK_BLK, _BW), lambda i: (i, 0)),
        ),
        out_shape=(
            jax.ShapeDtypeStruct((1, K), jnp.int32),
            jax.ShapeDtypeStruct((N, D), jnp.float32),
            jax.ShapeDtypeStruct((N, _BW), jnp.float32),
        ),
        scratch_shapes=[pltpu.VMEM((N, _BW), jnp.bfloat16)],
    )(s_row, s_col, h, g)


@functools.lru_cache(maxsize=None)
def _make_sc_gather(width, batch, rows_per_dma, dtype):
    info = plsc.get_sparse_core_info()
    nc, ns = info.num_cores, info.num_subcores
    nw = nc * ns
    b_per_w = batch // nw
    n_dma = b_per_w // rows_per_dma
    mesh = plsc.VectorSubcoreMesh(core_axis_name="c", subcore_axis_name="s")

    @functools.partial(
        pl.kernel,
        mesh=mesh,
        out_type=jax.ShapeDtypeStruct((batch, width), dtype),
        scratch_types=[
            pltpu.VMEM((b_per_w,), jnp.int32),
            pltpu.VMEM((rows_per_dma, width), dtype),
            pltpu.VMEM((rows_per_dma, width), dtype),
            pltpu.SemaphoreType.DMA,
            pltpu.SemaphoreType.DMA,
        ],
    )
    def k(table_hbm, idx_hbm, out_hbm, idx_v, buf0, buf1, sem0, sem1):
        wid = lax.axis_index("s") * nc + lax.axis_index("c")
        base = wid * b_per_w
        pltpu.sync_copy(idx_hbm.at[pl.ds(base, b_per_w)], idx_v)
        bufs, sems, cps = (buf0, buf1), (sem0, sem1), [None, None]
        r = rows_per_dma
        cps[0] = pltpu.async_copy(table_hbm.at[idx_v.at[pl.ds(0, r)]], bufs[0], sems[0])
        for j in range(n_dma):
            cur, nxt = j % 2, (j + 1) % 2
            if j + 1 < n_dma:
                cps[nxt] = pltpu.async_copy(
                    table_hbm.at[idx_v.at[pl.ds((j + 1) * r, r)]], bufs[nxt], sems[nxt]
                )
            cps[cur].wait()
            pltpu.sync_copy(bufs[cur], out_hbm.at[pl.ds(base + j * r, r)])

    return k


def _mm_body(idx_ref, arb_ref, bits_ref, out_ref, s_ref, e_ref):
    kk = pl.program_id(0)
    kblk = _MM_KBLK

    @pl.when(kk == 0)
    def _():
        out_ref[...] = jnp.zeros_like(out_ref)
        ci = lax.broadcasted_iota(jnp.int32, (_BW, K), 0)
        s_ref[...] = ((idx_ref[...] >> 3) == ci).astype(jnp.bfloat16)

    sel = lax.dot_general(
        bits_ref[...].astype(jnp.bfloat16), s_ref[...],
        (((1,), (0,)), ((), ())), preferred_element_type=jnp.float32,
    ).astype(jnp.int32)
    shj = idx_ref[...] & 7
    ac = ((sel >> shj) & 1).astype(jnp.bfloat16)
    ce = lax.broadcasted_iota(jnp.int32, (_BW, kblk), 0)
    ue = lax.broadcasted_iota(jnp.int32, (_BW, kblk), 1) + kk * kblk
    e_ref[...] = ((ue >> 3) == ce).astype(jnp.bfloat16)
    arx = lax.dot_general(
        arb_ref[...].astype(jnp.bfloat16), e_ref[...],
        (((1,), (0,)), ((), ())), preferred_element_type=jnp.float32,
    ).astype(jnp.int32)
    shu = lax.broadcasted_iota(jnp.int32, (K, kblk), 1) & 7
    ar = ((arx >> shu) & 1).astype(jnp.bfloat16)
    out_ref[...] += lax.dot_general(
        ar, ac, (((1,), (0,)), ((), ())), preferred_element_type=jnp.float32
    )

    @pl.when(kk == pl.num_programs(0) - 1)
    def _():
        u = (out_ref[...] != 0).astype(jnp.float32)
        deg = jnp.sum(u, axis=1, keepdims=True)
        dcol = lax.rsqrt(deg)
        drow = jnp.transpose(dcol)
        ii = lax.broadcasted_iota(jnp.int32, (K, K), 0)
        jj = lax.broadcasted_iota(jnp.int32, (K, K), 1)
        eye = (ii == jj).astype(jnp.float32)
        out_ref[...] = u * dcol * drow + eye


def _mm_call(idx2d, arb, bits):
    nblk = N // _MM_KBLK
    return pl.pallas_call(
        _mm_body,
        grid=(nblk,),
        in_specs=[
            pl.BlockSpec((1, K), lambda k: (0, 0)),
            pl.BlockSpec((K, _BW), lambda k: (0, 0)),
            pl.BlockSpec((_MM_KBLK, _BW), lambda k: (k, 0)),
        ],
        out_specs=pl.BlockSpec((K, K), lambda k: (0, 0)),
        out_shape=jax.ShapeDtypeStruct((K, K), jnp.float32),
        scratch_shapes=[
            pltpu.VMEM((_BW, K), jnp.bfloat16),
            pltpu.VMEM((_BW, _MM_KBLK), jnp.bfloat16),
        ],
    )(idx2d, arb, bits)


def kernel(g, h, W, b):
    _gather_h = _make_sc_gather(D, K, 64, jnp.float32)
    _gather_bits = _make_sc_gather(_BW, K, 64, jnp.float32)
    scores = jax.nn.sigmoid(jnp.squeeze(h @ W + b, -1))
    idx2d, hs, bits = _rank_pack_call(
        scores.reshape(1, N), scores.reshape(N, 1), h, g
    )
    idx = idx2d.reshape(K)
    new_h = _gather_h(hs, idx)
    arb = _gather_bits(bits, idx)
    g_new = _mm_call(idx2d, arb, bits)
    return (g_new, new_h, idx)


# speedup vs baseline: 3.1296x; 1.0611x over previous
import functools

import jax
import jax.numpy as jnp
from jax import lax
from jax.experimental import pallas as pl
from jax.experimental.pallas import tpu as pltpu
from jax.experimental.pallas import tpu_sc as plsc

N = 4096
D = 256
K = 2048

_RANK_BLK = 512
_MM_KBLK = 512
_NORM_BLK = 512


_PACK = 8
_BW = N // _PACK


def _rank_pack_body(srow_ref, scol_ref, h_ref, g_ref, idx_ref, hs_ref,
                    bits_ref, w8_ref):
    i = pl.program_id(0)

    @pl.when(i == 0)
    def _():
        idx_ref[...] = jnp.zeros_like(idx_ref)
        mi = lax.broadcasted_iota(jnp.int32, (N, _BW), 0)
        ci = lax.broadcasted_iota(jnp.int32, (N, _BW), 1)
        w8_ref[...] = jnp.where(
            (mi & (_BW - 1)) == ci, 1 << (mi // _BW), 0
        ).astype(jnp.bfloat16)

    sc = scol_ref[...]
    sr = srow_ref[...]
    jj = lax.broadcasted_iota(jnp.int32, (_RANK_BLK, N), 1)
    ii = lax.broadcasted_iota(jnp.int32, (_RANK_BLK, 1), 0) + i * _RANK_BLK
    beats = (sr > sc) | ((sr == sc) & (jj < ii))
    rank = jnp.sum(beats.astype(jnp.int32), axis=1, keepdims=True)
    pp = lax.broadcasted_iota(jnp.int32, (_RANK_BLK, K), 1)
    hit = rank == pp
    idx_ref[...] += jnp.sum(jnp.where(hit, ii, 0), axis=0, keepdims=True)
    hs_ref[...] = h_ref[...] * sc
    a = (g_ref[...] != 0).astype(jnp.bfloat16)
    bits_ref[...] = lax.dot_general(
        a, w8_ref[...], (((1,), (0,)), ((), ())),
        preferred_element_type=jnp.float32,
    )


def _rank_pack_call(s_row, s_col, h, g):
    nblk = N // _RANK_BLK
    return pl.pallas_call(
        _rank_pack_body,
        grid=(nblk,),
        in_specs=[
            pl.BlockSpec((1, N), lambda i: (0, 0)),
            pl.BlockSpec((_RANK_BLK, 1), lambda i: (i, 0)),
            pl.BlockSpec((_RANK_BLK, D), lambda i: (i, 0)),
            pl.BlockSpec((_RANK_BLK, N), lambda i: (i, 0)),
        ],
        out_specs=(
            pl.BlockSpec((1, K), lambda i: (0, 0)),
            pl.BlockSpec((_RANK_BLK, D), lambda i: (i, 0)),
            pl.BlockSpec((_RANK_BLK, _BW), lambda i: (i, 0)),
        ),
        out_shape=(
            jax.ShapeDtypeStruct((1, K), jnp.int32),
            jax.ShapeDtypeStruct((N, D), jnp.float32),
            jax.ShapeDtypeStruct((N, _BW), jnp.float32),
        ),
        scratch_shapes=[pltpu.VMEM((N, _BW), jnp.bfloat16)],
    )(s_row, s_col, h, g)


@functools.lru_cache(maxsize=None)
def _make_sc_gather(width, batch, rows_per_dma, dtype):
    info = plsc.get_sparse_core_info()
    nc, ns = info.num_cores, info.num_subcores
    nw = nc * ns
    b_per_w = batch // nw
    n_dma = b_per_w // rows_per_dma
    mesh = plsc.VectorSubcoreMesh(core_axis_name="c", subcore_axis_name="s")

    @functools.partial(
        pl.kernel,
        mesh=mesh,
        out_type=jax.ShapeDtypeStruct((batch, width), dtype),
        scratch_types=[
            pltpu.VMEM((b_per_w,), jnp.int32),
            pltpu.VMEM((rows_per_dma, width), dtype),
            pltpu.VMEM((rows_per_dma, width), dtype),
            pltpu.SemaphoreType.DMA,
            pltpu.SemaphoreType.DMA,
        ],
    )
    def k(table_hbm, idx_hbm, out_hbm, idx_v, buf0, buf1, sem0, sem1):
        wid = lax.axis_index("s") * nc + lax.axis_index("c")
        base = wid * b_per_w
        pltpu.sync_copy(idx_hbm.at[pl.ds(base, b_per_w)], idx_v)
        bufs, sems, cps = (buf0, buf1), (sem0, sem1), [None, None]
        r = rows_per_dma
        cps[0] = pltpu.async_copy(table_hbm.at[idx_v.at[pl.ds(0, r)]], bufs[0], sems[0])
        for j in range(n_dma):
            cur, nxt = j % 2, (j + 1) % 2
            if j + 1 < n_dma:
                cps[nxt] = pltpu.async_copy(
                    table_hbm.at[idx_v.at[pl.ds((j + 1) * r, r)]], bufs[nxt], sems[nxt]
                )
            cps[cur].wait()
            pltpu.sync_copy(bufs[cur], out_hbm.at[pl.ds(base + j * r, r)])

    return k


def _mm_body(idx_ref, arb_ref, bits_ref, out_ref, s_ref):
    kk = pl.program_id(0)

    @pl.when(kk == 0)
    def _():
        out_ref[...] = jnp.zeros_like(out_ref)
        ci = lax.broadcasted_iota(jnp.int32, (_BW, K), 0)
        s_ref[...] = ((idx_ref[...] & (_BW - 1)) == ci).astype(jnp.bfloat16)

    sel = lax.dot_general(
        bits_ref[...].astype(jnp.bfloat16), s_ref[...],
        (((1,), (0,)), ((), ())), preferred_element_type=jnp.float32,
    ).astype(jnp.int32)
    shj = idx_ref[...] // _BW
    ac = ((sel >> shj) & 1).astype(jnp.int8)
    ar = ((arb_ref[...].astype(jnp.int32) >> kk) & 1).astype(jnp.int8)
    out_ref[...] += lax.dot_general(
        ar, ac, (((1,), (0,)), ((), ())), preferred_element_type=jnp.int32
    ).astype(jnp.float32)

    @pl.when(kk == pl.num_programs(0) - 1)
    def _():
        u = (out_ref[...] != 0).astype(jnp.float32)
        deg = jnp.sum(u, axis=1, keepdims=True)
        dcol = lax.rsqrt(deg)
        drow = jnp.transpose(dcol)
        ii = lax.broadcasted_iota(jnp.int32, (K, K), 0)
        jj = lax.broadcasted_iota(jnp.int32, (K, K), 1)
        eye = (ii == jj).astype(jnp.float32)
        out_ref[...] = u * dcol * drow + eye


def _mm_call(idx2d, arb, bits):
    nblk = N // _MM_KBLK
    return pl.pallas_call(
        _mm_body,
        grid=(nblk,),
        in_specs=[
            pl.BlockSpec((1, K), lambda k: (0, 0)),
            pl.BlockSpec((K, _BW), lambda k: (0, 0)),
            pl.BlockSpec((_MM_KBLK, _BW), lambda k: (k, 0)),
        ],
        out_specs=pl.BlockSpec((K, K), lambda k: (0, 0)),
        out_shape=jax.ShapeDtypeStruct((K, K), jnp.float32),
        scratch_shapes=[pltpu.VMEM((_BW, K), jnp.bfloat16)],
    )(idx2d, arb, bits)


def kernel(g, h, W, b):
    _gather_h = _make_sc_gather(D, K, 64, jnp.float32)
    _gather_bits = _make_sc_gather(_BW, K, 64, jnp.float32)
    scores = jax.nn.sigmoid(h @ W + b)
    idx2d, hs, bits = _rank_pack_call(scores.reshape(1, N), scores, h, g)
    idx = idx2d.reshape(K)
    new_h = _gather_h(hs, idx)
    arb = _gather_bits(bits, idx)
    g_new = _mm_call(idx2d, arb, bits)
    return (g_new, new_h, idx)


# speedup vs baseline: 3.1668x; 1.0119x over previous
import functools

import jax
import jax.numpy as jnp
from jax import lax
from jax.experimental import pallas as pl
from jax.experimental.pallas import tpu as pltpu
from jax.experimental.pallas import tpu_sc as plsc

N = 4096
D = 256
K = 2048

_RANK_BLK = 512
_MM_KBLK = 512
_NORM_BLK = 512


_PACK = 8
_BW = N // _PACK


def _rank_pack_body(srow_ref, scol_ref, h_ref, g_ref, idx_ref, hs_ref,
                    bits_ref, w8_ref):
    i = pl.program_id(0)

    @pl.when(i == 0)
    def _():
        idx_ref[...] = jnp.zeros_like(idx_ref)
        mi = lax.broadcasted_iota(jnp.int32, (N, _BW), 0)
        ci = lax.broadcasted_iota(jnp.int32, (N, _BW), 1)
        w8_ref[...] = jnp.where(
            (mi & (_BW - 1)) == ci, 1 << (mi // _BW), 0
        ).astype(jnp.bfloat16)

    sc = scol_ref[...]
    sr = srow_ref[...]
    jj = lax.broadcasted_iota(jnp.int32, (_RANK_BLK, N), 1)
    ii = lax.broadcasted_iota(jnp.int32, (_RANK_BLK, 1), 0) + i * _RANK_BLK
    beats = (sr > sc) | ((sr == sc) & (jj < ii))
    rank = jnp.sum(beats.astype(jnp.int32), axis=1, keepdims=True)
    pp = lax.broadcasted_iota(jnp.int32, (_RANK_BLK, K), 1)
    hit = rank == pp
    idx_ref[...] += jnp.sum(jnp.where(hit, ii, 0), axis=0, keepdims=True)
    hs_ref[...] = h_ref[...] * sc
    a = (g_ref[...] != 0).astype(jnp.bfloat16)
    bits_ref[...] = lax.dot_general(
        a, w8_ref[...], (((1,), (0,)), ((), ())),
        preferred_element_type=jnp.float32,
    )


def _rank_pack_call(s_row, s_col, h, g):
    nblk = N // _RANK_BLK
    return pl.pallas_call(
        _rank_pack_body,
        grid=(nblk,),
        in_specs=[
            pl.BlockSpec((1, N), lambda i: (0, 0)),
            pl.BlockSpec((_RANK_BLK, 1), lambda i: (i, 0)),
            pl.BlockSpec((_RANK_BLK, D), lambda i: (i, 0)),
            pl.BlockSpec((_RANK_BLK, N), lambda i: (i, 0)),
        ],
        out_specs=(
            pl.BlockSpec((1, K), lambda i: (0, 0)),
            pl.BlockSpec((_RANK_BLK, D), lambda i: (i, 0)),
            pl.BlockSpec((_RANK_BLK, _BW), lambda i: (i, 0)),
        ),
        out_shape=(
            jax.ShapeDtypeStruct((1, K), jnp.int32),
            jax.ShapeDtypeStruct((N, D), jnp.float32),
            jax.ShapeDtypeStruct((N, _BW), jnp.float32),
        ),
        scratch_shapes=[pltpu.VMEM((N, _BW), jnp.bfloat16)],
    )(s_row, s_col, h, g)


@functools.lru_cache(maxsize=None)
def _make_sc_gather(width, batch, rows_per_dma, dtype):
    info = plsc.get_sparse_core_info()
    nc, ns = info.num_cores, info.num_subcores
    nw = nc * ns
    b_per_w = batch // nw
    n_dma = b_per_w // rows_per_dma
    mesh = plsc.VectorSubcoreMesh(core_axis_name="c", subcore_axis_name="s")

    @functools.partial(
        pl.kernel,
        mesh=mesh,
        out_type=jax.ShapeDtypeStruct((batch, width), dtype),
        scratch_types=[
            pltpu.VMEM((b_per_w,), jnp.int32),
            pltpu.VMEM((rows_per_dma, width), dtype),
            pltpu.VMEM((rows_per_dma, width), dtype),
            pltpu.SemaphoreType.DMA,
            pltpu.SemaphoreType.DMA,
        ],
    )
    def k(table_hbm, idx_hbm, out_hbm, idx_v, buf0, buf1, sem0, sem1):
        wid = lax.axis_index("s") * nc + lax.axis_index("c")
        base = wid * b_per_w
        pltpu.sync_copy(idx_hbm.at[pl.ds(base, b_per_w)], idx_v)
        bufs, sems, cps = (buf0, buf1), (sem0, sem1), [None, None]
        r = rows_per_dma
        cps[0] = pltpu.async_copy(table_hbm.at[idx_v.at[pl.ds(0, r)]], bufs[0], sems[0])
        for j in range(n_dma):
            cur, nxt = j % 2, (j + 1) % 2
            if j + 1 < n_dma:
                cps[nxt] = pltpu.async_copy(
                    table_hbm.at[idx_v.at[pl.ds((j + 1) * r, r)]], bufs[nxt], sems[nxt]
                )
            cps[cur].wait()
            pltpu.sync_copy(bufs[cur], out_hbm.at[pl.ds(base + j * r, r)])

    return k


def _mm_body(idx_ref, arb_ref, bits_ref, out_ref, s_ref):
    kk = pl.program_id(0)

    @pl.when(kk == 0)
    def _():
        out_ref[...] = jnp.zeros_like(out_ref)
        ci = lax.broadcasted_iota(jnp.int32, (_BW, K), 0)
        s_ref[...] = ((idx_ref[...] & (_BW - 1)) == ci).astype(jnp.bfloat16)

    sel = lax.dot_general(
        bits_ref[...].astype(jnp.bfloat16), s_ref[...],
        (((1,), (0,)), ((), ())), preferred_element_type=jnp.float32,
    ).astype(jnp.int32)
    shj = idx_ref[...] // _BW
    ac = ((sel >> shj) & 1).astype(jnp.int8)
    ar = ((arb_ref[...].astype(jnp.int32) >> kk) & 1).astype(jnp.int8)
    acc = lax.bitcast_convert_type(out_ref[...], jnp.int32) + lax.dot_general(
        ar, ac, (((1,), (0,)), ((), ())), preferred_element_type=jnp.int32
    )
    out_ref[...] = lax.bitcast_convert_type(acc, jnp.float32)

    @pl.when(kk == pl.num_programs(0) - 1)
    def _():
        cnt = lax.bitcast_convert_type(out_ref[...], jnp.int32)
        u = (cnt != 0).astype(jnp.float32)
        deg = jnp.sum(u, axis=1, keepdims=True)
        dcol = lax.rsqrt(deg)
        drow = jnp.transpose(dcol)
        ii = lax.broadcasted_iota(jnp.int32, (K, K), 0)
        jj = lax.broadcasted_iota(jnp.int32, (K, K), 1)
        eye = (ii == jj).astype(jnp.float32)
        out_ref[...] = u * dcol * drow + eye


def _mm_call(idx2d, arb, bits):
    nblk = N // _MM_KBLK
    return pl.pallas_call(
        _mm_body,
        grid=(nblk,),
        in_specs=[
            pl.BlockSpec((1, K), lambda k: (0, 0)),
            pl.BlockSpec((K, _BW), lambda k: (0, 0)),
            pl.BlockSpec((_MM_KBLK, _BW), lambda k: (k, 0)),
        ],
        out_specs=pl.BlockSpec((K, K), lambda k: (0, 0)),
        out_shape=jax.ShapeDtypeStruct((K, K), jnp.float32),
        scratch_shapes=[pltpu.VMEM((_BW, K), jnp.bfloat16)],
    )(idx2d, arb, bits)


def kernel(g, h, W, b):
    _gather_h = _make_sc_gather(D, K, 64, jnp.float32)
    _gather_bits = _make_sc_gather(_BW, K, 64, jnp.float32)
    scores = jax.nn.sigmoid(h @ W + b)
    idx2d, hs, bits = _rank_pack_call(scores.reshape(1, N), scores, h, g)
    idx = idx2d.reshape(K)
    new_h = _gather_h(hs, idx)
    arb = _gather_bits(bits, idx)
    g_new = _mm_call(idx2d, arb, bits)
    return (g_new, new_h, idx)
